# baseline (speedup 1.0000x reference)
import functools

import jax
import jax.numpy as jnp
from jax import lax
from jax.experimental import pallas as pl
from jax.experimental.pallas import tpu as pltpu
from jax.experimental.pallas import tpu_sc as plsc

N = 10000
E = 320000
F_IN = 128
H = 256
G = 64
C_OUT = 2
BN_EPS = 1e-5

NBLK = 5
BLK = N // NBLK
K_EDGE = 48
RING = 6
NTILES = 16
N_PAD = 10240
ROWS_PER_TILE = N_PAD // NTILES
HALF = H // 2


def _sc_edge_loop(feat_hbm, eidx_hbm, idxv, rows, acc, isems, gsems, ssems,
                  nchunks):
    def i_start(s, ch):
        pltpu.async_copy(eidx_hbm.at[ch], idxv.at[s], isems[s])

    def i_wait(s):
        pltpu.make_async_copy(eidx_hbm.at[0], idxv.at[s], isems[s]).wait()

    def g_start(s):
        pltpu.async_copy(feat_hbm.at[idxv.at[s, 0]], rows.at[s], gsems[s])

    def g_wait(s):
        pltpu.make_async_copy(feat_hbm.at[idxv.at[0, 0]], rows.at[s],
                              gsems[s]).wait()

    def s_start(s):
        pltpu.async_copy(rows.at[s], acc.at[idxv.at[s, 1]], ssems[s], add=True)

    def s_wait(s):
        pltpu.make_async_copy(rows.at[s], acc.at[idxv.at[0, 1]],
                              ssems[s]).wait()

    def refill(s, c):
        @pl.when(c < nchunks)
        def _():
            i_start(s, c)
            i_wait(s)
            g_start(s)

    assert nchunks >= RING + 1
    for s in range(RING):
        i_start(s, s)
    for s in range(RING):
        i_wait(s)
        g_start(s)

    def body(i, carry):
        ch = RING * i
        for s in range(RING):
            g_wait(s)
            s_start(s)
        for s in range(RING):
            s_wait(s)
            refill(s, ch + RING + s)
        return carry

    lax.fori_loop(0, nchunks // RING, body, 0)
    rem = nchunks % RING
    for s in range(rem):
        g_wait(s)
        s_start(s)
    for s in range(rem):
        s_wait(s)


def _agg_l1(x, eidx4, zrows):
    nchunks = eidx4.shape[1]
    mesh = plsc.VectorSubcoreMesh(core_axis_name="c", subcore_axis_name="s")

    @functools.partial(
        pl.kernel,
        out_type=jax.ShapeDtypeStruct((2, N_PAD, F_IN), jnp.float32),
        mesh=mesh,
        scratch_types=[
            pltpu.VMEM((RING, 2, K_EDGE), jnp.int32),
            pltpu.VMEM((RING, K_EDGE, F_IN), jnp.float32),
            pltpu.VMEM_SHARED((N_PAD, F_IN), jnp.float32),
        ] + [pltpu.SemaphoreType.DMA] * (3 * RING),
    )
    def k(x_hbm, eidx_hbm, z_hbm, out_hbm, idxv, rows, acc, *sems):
        cid = lax.axis_index("c")
        sid = lax.axis_index("s")
        w = cid * NTILES + sid
        pltpu.sync_copy(z_hbm, acc.at[pl.ds(sid * ROWS_PER_TILE, ROWS_PER_TILE)])
        plsc.subcore_barrier()
        _sc_edge_loop(x_hbm, eidx_hbm.at[w], idxv, rows, acc,
                      sems[0:RING], sems[RING:2 * RING], sems[2 * RING:3 * RING], nchunks)
        plsc.subcore_barrier()
        pltpu.sync_copy(
            acc.at[pl.ds(sid * ROWS_PER_TILE, ROWS_PER_TILE)],
            out_hbm.at[cid, pl.ds(sid * ROWS_PER_TILE, ROWS_PER_TILE)])

    return k(x, eidx4, zrows)


def _agg_l23(h, eidx4, zrows):
    nchunks = eidx4.shape[1]
    mesh = plsc.VectorSubcoreMesh(core_axis_name="c", subcore_axis_name="s")

    @functools.partial(
        pl.kernel,
        out_type=jax.ShapeDtypeStruct((2, N_PAD, HALF), jnp.float32),
        mesh=mesh,
        scratch_types=[
            pltpu.VMEM((RING, 2, K_EDGE), jnp.int32),
            pltpu.VMEM((RING, K_EDGE, HALF), jnp.float32),
            pltpu.VMEM_SHARED((N_PAD, HALF), jnp.float32),
        ] + [pltpu.SemaphoreType.DMA] * (3 * RING),
    )
    def k(h_hbm, eidx_hbm, z_hbm, out_hbm, idxv, rows, acc, *sems):
        cid = lax.axis_index("c")
        sid = lax.axis_index("s")
        pltpu.sync_copy(z_hbm, acc.at[pl.ds(sid * ROWS_PER_TILE, ROWS_PER_TILE)])
        plsc.subcore_barrier()
        _sc_edge_loop(h_hbm.at[cid], eidx_hbm.at[sid], idxv, rows, acc,
                      sems[0:RING], sems[RING:2 * RING], sems[2 * RING:3 * RING], nchunks)
        plsc.subcore_barrier()
        pltpu.sync_copy(
            acc.at[pl.ds(sid * ROWS_PER_TILE, ROWS_PER_TILE)],
            out_hbm.at[cid, pl.ds(sid * ROWS_PER_TILE, ROWS_PER_TILE)])

    return k(h, eidx4, zrows)


def _pad_tables(edge_index, nworkers):
    per_worker = E // nworkers
    per_padded = -(-per_worker // K_EDGE) * K_EDGE
    npad = per_padded - per_worker
    e3 = edge_index.reshape(2, nworkers, per_worker)
    if npad:
        pad_dst = N + 16 + (jnp.arange(npad, dtype=jnp.int32) % (N_PAD - N - 64))
        pad = jnp.stack([
            jnp.zeros((npad,), jnp.int32),
            pad_dst,
        ])[:, None, :].repeat(nworkers, axis=1)
        cat = jnp.concatenate([e3, pad], axis=2)
    else:
        cat = e3
    return cat.reshape(2, nworkers, per_padded // K_EDGE,
                       K_EDGE).transpose(1, 2, 0, 3)


def _gin_phase1(y, su_ref, sq_ref, gm_ref, bt_ref, w2_ref, b2_ref, bat_ref,
                h_ref, p_ref, i):
    mu = su_ref[...] * (1.0 / N)
    var = sq_ref[...] * (1.0 / N) - mu * mu
    inv = lax.rsqrt(var + BN_EPS)
    scale = gm_ref[...] * inv
    shift = bt_ref[...] - mu * scale
    z = jnp.maximum(y * scale + shift, 0.0)
    h = jnp.maximum(
        jnp.dot(z, w2_ref[...], preferred_element_type=jnp.float32) + b2_ref[...],
        0.0)
    h_ref[0] = h[:, :HALF]
    h_ref[1] = h[:, HALF:]

    bat = bat_ref[0]
    gids = lax.broadcasted_iota(jnp.int32, (G, BLK), 0)
    oh = jnp.where(gids == bat, 1.0, 0.0)

    @pl.when(i == 0)
    def _():
        p_ref[...] = jnp.zeros_like(p_ref)

    p_ref[0] += jnp.dot(oh, h[:, :HALF], preferred_element_type=jnp.float32)
    p_ref[1] += jnp.dot(oh, h[:, HALF:], preferred_element_type=jnp.float32)


def _accum_stats(y, su_ref, sq_ref, i):
    @pl.when(i == 0)
    def _():
        su_ref[...] = jnp.zeros_like(su_ref)
        sq_ref[...] = jnp.zeros_like(sq_ref)

    su_ref[...] += jnp.sum(y, axis=0, keepdims=True)
    sq_ref[...] += jnp.sum(y * y, axis=0, keepdims=True)


def _gin_l1_body(x_ref, aa_ref, ab_ref, w_ref, b_ref, gm_ref, bt_ref, w2_ref,
                 b2_ref, bat_ref, h_ref, p_ref, y_scr, su_ref, sq_ref):
    p = pl.program_id(0)
    i = pl.program_id(1)

    @pl.when(p == 0)
    def _():
        xin = x_ref[...] + aa_ref[0] + ab_ref[0]
        y = (jnp.dot(xin, w_ref[...], preferred_element_type=jnp.float32)
             + b_ref[...])
        y_scr[pl.ds(i * BLK, BLK), :] = y
        _accum_stats(y, su_ref, sq_ref, i)

    @pl.when(p == 1)
    def _():
        _gin_phase1(y_scr[pl.ds(i * BLK, BLK), :], su_ref, sq_ref, gm_ref,
                    bt_ref, w2_ref, b2_ref, bat_ref, h_ref, p_ref, i)


def _gin_l1(x, agg, c, batch3):
    return pl.pallas_call(
        _gin_l1_body,
        grid=(2, NBLK),
        in_specs=[
            pl.BlockSpec((BLK, F_IN), lambda p, i: (i * (1 - p), 0)),
            pl.BlockSpec((1, BLK, F_IN), lambda p, i: (0, i * (1 - p), 0)),
            pl.BlockSpec((1, BLK, F_IN), lambda p, i: (1, i * (1 - p), 0)),
            pl.BlockSpec((F_IN, H), lambda p, i: (0, 0)),
            pl.BlockSpec((1, H), lambda p, i: (0, 0)),
            pl.BlockSpec((1, H), lambda p, i: (0, 0)),
            pl.BlockSpec((1, H), lambda p, i: (0, 0)),
            pl.BlockSpec((H, H), lambda p, i: (0, 0)),
            pl.BlockSpec((1, H), lambda p, i: (0, 0)),
            pl.BlockSpec((1, 1, BLK), lambda p, i: (i * p, 0, 0)),
        ],
        out_specs=[
            pl.BlockSpec((2, BLK, HALF), lambda p, i: (0, i * p, 0)),
            pl.BlockSpec((2, G, HALF), lambda p, i: (0, 0, 0)),
        ],
        out_shape=[
            jax.ShapeDtypeStruct((2, N_PAD, HALF), jnp.float32),
            jax.ShapeDtypeStruct((2, G, HALF), jnp.float32),
        ],
        scratch_shapes=[
            pltpu.VMEM((N, H), jnp.float32),
            pltpu.VMEM((1, H), jnp.float32),
            pltpu.VMEM((1, H), jnp.float32),
        ],
    )(x, agg, agg, c['W1'], c['b1'].reshape(1, H), c['gamma'].reshape(1, H),
      c['beta'].reshape(1, H), c['W2'], c['b2'].reshape(1, H), batch3)


def _gin_l23_body(xs_ref, as_ref, w_ref, b_ref, gm_ref, bt_ref, w2_ref,
                  b2_ref, bat_ref, h_ref, p_ref, y_scr, su_ref, sq_ref):
    p = pl.program_id(0)
    i = pl.program_id(1)

    @pl.when(p == 0)
    def _():
        lo = xs_ref[0] + as_ref[0]
        hi = xs_ref[1] + as_ref[1]
        y = (jnp.dot(lo, w_ref[:HALF], preferred_element_type=jnp.float32)
             + jnp.dot(hi, w_ref[HALF:], preferred_element_type=jnp.float32)
             + b_ref[...])
        y_scr[pl.ds(i * BLK, BLK), :] = y
        _accum_stats(y, su_ref, sq_ref, i)

    @pl.when(p == 1)
    def _():
        _gin_phase1(y_scr[pl.ds(i * BLK, BLK), :], su_ref, sq_ref, gm_ref,
                    bt_ref, w2_ref, b2_ref, bat_ref, h_ref, p_ref, i)


def _gin_l23(hs, aggs, c, batch3):
    return pl.pallas_call(
        _gin_l23_body,
        grid=(2, NBLK),
        in_specs=[
            pl.BlockSpec((2, BLK, HALF), lambda p, i: (0, i * (1 - p), 0)),
            pl.BlockSpec((2, BLK, HALF), lambda p, i: (0, i * (1 - p), 0)),
            pl.BlockSpec((H, H), lambda p, i: (0, 0)),
            pl.BlockSpec((1, H), lambda p, i: (0, 0)),
            pl.BlockSpec((1, H), lambda p, i: (0, 0)),
            pl.BlockSpec((1, H), lambda p, i: (0, 0)),
            pl.BlockSpec((H, H), lambda p, i: (0, 0)),
            pl.BlockSpec((1, H), lambda p, i: (0, 0)),
            pl.BlockSpec((1, 1, BLK), lambda p, i: (i * p, 0, 0)),
        ],
        out_specs=[
            pl.BlockSpec((2, BLK, HALF), lambda p, i: (0, i * p, 0)),
            pl.BlockSpec((2, G, HALF), lambda p, i: (0, 0, 0)),
        ],
        out_shape=[
            jax.ShapeDtypeStruct((2, N_PAD, HALF), jnp.float32),
            jax.ShapeDtypeStruct((2, G, HALF), jnp.float32),
        ],
        scratch_shapes=[
            pltpu.VMEM((N, H), jnp.float32),
            pltpu.VMEM((1, H), jnp.float32),
            pltpu.VMEM((1, H), jnp.float32),
        ],
    )(hs, aggs, c['W1'], c['b1'].reshape(1, H), c['gamma'].reshape(1, H),
      c['beta'].reshape(1, H), c['W2'], c['b2'].reshape(1, H), batch3)


def _cls_body(p_ref, w1_ref, b1_ref, w2_ref, b2_ref, o_ref):
    acc = jnp.zeros((G, 3 * H), jnp.float32)
    for kk in range(6):
        acc = acc + jnp.dot(p_ref[kk], w1_ref[kk],
                            preferred_element_type=jnp.float32)
    hh = jnp.maximum(acc + b1_ref[...], 0.0)
    lg = jnp.dot(hh, w2_ref[...], preferred_element_type=jnp.float32) + b2_ref[...]
    m = jnp.max(lg, axis=1, keepdims=True)
    lse = m + jnp.log(jnp.sum(jnp.exp(lg - m), axis=1, keepdims=True))
    o_ref[...] = lg - lse


def _classifier(pcat, w1r, b1, w2p, b2p):
    return pl.pallas_call(
        _cls_body,
        grid=(1,),
        in_specs=[
            pl.BlockSpec((6, G, HALF), lambda i: (0, 0, 0)),
            pl.BlockSpec((6, HALF, 3 * H), lambda i: (0, 0, 0)),
            pl.BlockSpec((1, 3 * H), lambda i: (0, 0)),
            pl.BlockSpec((3 * H, 128), lambda i: (0, 0)),
            pl.BlockSpec((1, 128), lambda i: (0, 0)),
        ],
        out_specs=pl.BlockSpec((G, 128), lambda i: (0, 0)),
        out_shape=jax.ShapeDtypeStruct((G, 128), jnp.float32),
    )(pcat, w1r, b1, w2p, b2p)


def kernel(x, edge_index, batch, params):
    eidx32 = _pad_tables(edge_index, 2 * NTILES)
    eidx16 = _pad_tables(edge_index, NTILES)
    zrows = jnp.zeros((ROWS_PER_TILE, HALF), jnp.float32)
    batch3 = batch.reshape(NBLK, 1, BLK)
    c1, c2, c3 = params['c1'], params['c2'], params['c3']

    agg1 = _agg_l1(x, eidx32, zrows)
    h1, p1 = _gin_l1(x, agg1, c1, batch3)

    agg2 = _agg_l23(h1, eidx16, zrows)
    h2, p2 = _gin_l23(h1, agg2, c2, batch3)

    agg3 = _agg_l23(h2, eidx16, zrows)
    h3, p3 = _gin_l23(h2, agg3, c3, batch3)

    pcat = jnp.concatenate([p1, p2, p3], axis=0)
    w1r = params['lin1_W'].reshape(6, HALF, 3 * H)
    b1r = params['lin1_b'].reshape(1, 3 * H)
    w2p = jnp.pad(params['lin2_W'], ((0, 0), (0, 128 - C_OUT)))
    b2p = jnp.concatenate(
        [params['lin2_b'], jnp.full((128 - C_OUT,), -1e9, jnp.float32)]
    ).reshape(1, 128)
    out = _classifier(pcat, w1r, b1r, w2p, b2p)
    return out[:, :C_OUT]


# speedup vs baseline: 1.3435x; 1.3435x over previous
import functools

import jax
import jax.numpy as jnp
from jax import lax
from jax.experimental import pallas as pl
from jax.experimental.pallas import tpu as pltpu
from jax.experimental.pallas import tpu_sc as plsc

N = 10000
E = 320000
F_IN = 128
H = 256
G = 64
C_OUT = 2
BN_EPS = 1e-5

NBLK = 5
BLK = N // NBLK
K_EDGE = 80
RING = 4
NTILES = 16
N_PAD = 10240
ROWS_PER_TILE = N_PAD // NTILES
HALF = H // 2


def _sc_edge_loop(feat_hbm, eidx_hbm, idxv, rows, acc, isems, gsems, ssems,
                  nchunks):
    def i_start(s, ch):
        pltpu.async_copy(eidx_hbm.at[ch], idxv.at[s], isems[s])

    def i_wait(s):
        pltpu.make_async_copy(eidx_hbm.at[0], idxv.at[s], isems[s]).wait()

    def g_start(s):
        pltpu.async_copy(feat_hbm.at[idxv.at[s, 0]], rows.at[s], gsems[s])

    def g_wait(s):
        pltpu.make_async_copy(feat_hbm.at[idxv.at[0, 0]], rows.at[s],
                              gsems[s]).wait()

    def s_start(s):
        pltpu.async_copy(rows.at[s], acc.at[idxv.at[s, 1]], ssems[s], add=True)

    def s_wait(s):
        pltpu.make_async_copy(rows.at[s], acc.at[idxv.at[0, 1]],
                              ssems[s]).wait()

    def refill(s, c):
        @pl.when(c < nchunks)
        def _():
            i_start(s, c)
            i_wait(s)
            g_start(s)

    assert nchunks >= RING + 1
    for s in range(RING):
        i_start(s, s)
    for s in range(RING):
        i_wait(s)
        g_start(s)

    def body(i, carry):
        ch = RING * i
        for s in range(RING):
            g_wait(s)
            s_start(s)
        for s in range(RING):
            s_wait(s)
            refill(s, ch + RING + s)
        return carry

    lax.fori_loop(0, nchunks // RING, body, 0)
    rem = nchunks % RING
    for s in range(rem):
        g_wait(s)
        s_start(s)
    for s in range(rem):
        s_wait(s)


def _agg_l1(x, eidx4, zrows):
    nchunks = eidx4.shape[1]
    mesh = plsc.VectorSubcoreMesh(core_axis_name="c", subcore_axis_name="s")

    @functools.partial(
        pl.kernel,
        out_type=jax.ShapeDtypeStruct((2, N_PAD, F_IN), jnp.float32),
        mesh=mesh,
        scratch_types=[
            pltpu.VMEM((RING, 2, K_EDGE), jnp.int32),
            pltpu.VMEM((RING, K_EDGE, F_IN), jnp.float32),
            pltpu.VMEM_SHARED((N_PAD, F_IN), jnp.float32),
        ] + [pltpu.SemaphoreType.DMA] * (3 * RING),
    )
    def k(x_hbm, eidx_hbm, z_hbm, out_hbm, idxv, rows, acc, *sems):
        cid = lax.axis_index("c")
        sid = lax.axis_index("s")
        w = cid * NTILES + sid
        pltpu.sync_copy(z_hbm, acc.at[pl.ds(sid * ROWS_PER_TILE, ROWS_PER_TILE)])
        plsc.subcore_barrier()
        _sc_edge_loop(x_hbm, eidx_hbm.at[w], idxv, rows, acc,
                      sems[0:RING], sems[RING:2 * RING], sems[2 * RING:3 * RING], nchunks)
        plsc.subcore_barrier()
        pltpu.sync_copy(
            acc.at[pl.ds(sid * ROWS_PER_TILE, ROWS_PER_TILE)],
            out_hbm.at[cid, pl.ds(sid * ROWS_PER_TILE, ROWS_PER_TILE)])

    return k(x, eidx4, zrows)


def _agg_l23(h, eidx4, zrows):
    nchunks = eidx4.shape[1]
    mesh = plsc.VectorSubcoreMesh(core_axis_name="c", subcore_axis_name="s")

    @functools.partial(
        pl.kernel,
        out_type=jax.ShapeDtypeStruct((2, N_PAD, HALF), jnp.float32),
        mesh=mesh,
        scratch_types=[
            pltpu.VMEM((RING, 2, K_EDGE), jnp.int32),
            pltpu.VMEM((RING, K_EDGE, HALF), jnp.float32),
            pltpu.VMEM_SHARED((N_PAD, HALF), jnp.float32),
        ] + [pltpu.SemaphoreType.DMA] * (3 * RING),
    )
    def k(h_hbm, eidx_hbm, z_hbm, out_hbm, idxv, rows, acc, *sems):
        cid = lax.axis_index("c")
        sid = lax.axis_index("s")
        pltpu.sync_copy(z_hbm, acc.at[pl.ds(sid * ROWS_PER_TILE, ROWS_PER_TILE)])
        plsc.subcore_barrier()
        _sc_edge_loop(h_hbm.at[cid], eidx_hbm.at[sid], idxv, rows, acc,
                      sems[0:RING], sems[RING:2 * RING], sems[2 * RING:3 * RING], nchunks)
        plsc.subcore_barrier()
        pltpu.sync_copy(
            acc.at[pl.ds(sid * ROWS_PER_TILE, ROWS_PER_TILE)],
            out_hbm.at[cid, pl.ds(sid * ROWS_PER_TILE, ROWS_PER_TILE)])

    return k(h, eidx4, zrows)


def _pad_tables(edge_index, nworkers):
    per_worker = E // nworkers
    per_padded = -(-per_worker // K_EDGE) * K_EDGE
    npad = per_padded - per_worker
    e3 = edge_index.reshape(2, nworkers, per_worker)
    if npad:
        pad_dst = N + 16 + (jnp.arange(npad, dtype=jnp.int32) % (N_PAD - N - 64))
        pad = jnp.stack([
            jnp.zeros((npad,), jnp.int32),
            pad_dst,
        ])[:, None, :].repeat(nworkers, axis=1)
        cat = jnp.concatenate([e3, pad], axis=2)
    else:
        cat = e3
    return cat.reshape(2, nworkers, per_padded // K_EDGE,
                       K_EDGE).transpose(1, 2, 0, 3)


def _gin_phase1(y, su_ref, sq_ref, gm_ref, bt_ref, w2_ref, b2_ref, bat_ref,
                h_ref, p_ref, i):
    mu = su_ref[...] * (1.0 / N)
    var = sq_ref[...] * (1.0 / N) - mu * mu
    inv = lax.rsqrt(var + BN_EPS)
    scale = gm_ref[...] * inv
    shift = bt_ref[...] - mu * scale
    z = jnp.maximum(y * scale + shift, 0.0)
    h = jnp.maximum(
        jnp.dot(z, w2_ref[...], preferred_element_type=jnp.float32) + b2_ref[...],
        0.0)
    h_ref[0] = h[:, :HALF]
    h_ref[1] = h[:, HALF:]

    bat = bat_ref[0]
    gids = lax.broadcasted_iota(jnp.int32, (G, BLK), 0)
    oh = jnp.where(gids == bat, 1.0, 0.0)

    @pl.when(i == 0)
    def _():
        p_ref[...] = jnp.zeros_like(p_ref)

    p_ref[0] += jnp.dot(oh, h[:, :HALF], preferred_element_type=jnp.float32)
    p_ref[1] += jnp.dot(oh, h[:, HALF:], preferred_element_type=jnp.float32)


def _accum_stats(y, su_ref, sq_ref, i):
    @pl.when(i == 0)
    def _():
        su_ref[...] = jnp.zeros_like(su_ref)
        sq_ref[...] = jnp.zeros_like(sq_ref)

    su_ref[...] += jnp.sum(y, axis=0, keepdims=True)
    sq_ref[...] += jnp.sum(y * y, axis=0, keepdims=True)


def _gin_l1_body(x_ref, aa_ref, ab_ref, w_ref, b_ref, gm_ref, bt_ref, w2_ref,
                 b2_ref, bat_ref, h_ref, p_ref, y_scr, su_ref, sq_ref):
    p = pl.program_id(0)
    i = pl.program_id(1)

    @pl.when(p == 0)
    def _():
        xin = x_ref[...] + aa_ref[0] + ab_ref[0]
        y = (jnp.dot(xin, w_ref[...], preferred_element_type=jnp.float32)
             + b_ref[...])
        y_scr[pl.ds(i * BLK, BLK), :] = y
        _accum_stats(y, su_ref, sq_ref, i)

    @pl.when(p == 1)
    def _():
        _gin_phase1(y_scr[pl.ds(i * BLK, BLK), :], su_ref, sq_ref, gm_ref,
                    bt_ref, w2_ref, b2_ref, bat_ref, h_ref, p_ref, i)


def _gin_l1(x, agg, c, batch3):
    return pl.pallas_call(
        _gin_l1_body,
        grid=(2, NBLK),
        in_specs=[
            pl.BlockSpec((BLK, F_IN), lambda p, i: (i * (1 - p), 0)),
            pl.BlockSpec((1, BLK, F_IN), lambda p, i: (0, i * (1 - p), 0)),
            pl.BlockSpec((1, BLK, F_IN), lambda p, i: (1, i * (1 - p), 0)),
            pl.BlockSpec((F_IN, H), lambda p, i: (0, 0)),
            pl.BlockSpec((1, H), lambda p, i: (0, 0)),
            pl.BlockSpec((1, H), lambda p, i: (0, 0)),
            pl.BlockSpec((1, H), lambda p, i: (0, 0)),
            pl.BlockSpec((H, H), lambda p, i: (0, 0)),
            pl.BlockSpec((1, H), lambda p, i: (0, 0)),
            pl.BlockSpec((1, 1, BLK), lambda p, i: (i * p, 0, 0)),
        ],
        out_specs=[
            pl.BlockSpec((2, BLK, HALF), lambda p, i: (0, i * p, 0)),
            pl.BlockSpec((2, G, HALF), lambda p, i: (0, 0, 0)),
        ],
        out_shape=[
            jax.ShapeDtypeStruct((2, N_PAD, HALF), jnp.float32),
            jax.ShapeDtypeStruct((2, G, HALF), jnp.float32),
        ],
        scratch_shapes=[
            pltpu.VMEM((N, H), jnp.float32),
            pltpu.VMEM((1, H), jnp.float32),
            pltpu.VMEM((1, H), jnp.float32),
        ],
    )(x, agg, agg, c['W1'], c['b1'].reshape(1, H), c['gamma'].reshape(1, H),
      c['beta'].reshape(1, H), c['W2'], c['b2'].reshape(1, H), batch3)


def _gin_l23_body(xs_ref, as_ref, w_ref, b_ref, gm_ref, bt_ref, w2_ref,
                  b2_ref, bat_ref, h_ref, p_ref, y_scr, su_ref, sq_ref):
    p = pl.program_id(0)
    i = pl.program_id(1)

    @pl.when(p == 0)
    def _():
        lo = xs_ref[0] + as_ref[0]
        hi = xs_ref[1] + as_ref[1]
        y = (jnp.dot(lo, w_ref[:HALF], preferred_element_type=jnp.float32)
             + jnp.dot(hi, w_ref[HALF:], preferred_element_type=jnp.float32)
             + b_ref[...])
        y_scr[pl.ds(i * BLK, BLK), :] = y
        _accum_stats(y, su_ref, sq_ref, i)

    @pl.when(p == 1)
    def _():
        _gin_phase1(y_scr[pl.ds(i * BLK, BLK), :], su_ref, sq_ref, gm_ref,
                    bt_ref, w2_ref, b2_ref, bat_ref, h_ref, p_ref, i)


def _gin_l23(hs, aggs, c, batch3):
    return pl.pallas_call(
        _gin_l23_body,
        grid=(2, NBLK),
        in_specs=[
            pl.BlockSpec((2, BLK, HALF), lambda p, i: (0, i * (1 - p), 0)),
            pl.BlockSpec((2, BLK, HALF), lambda p, i: (0, i * (1 - p), 0)),
            pl.BlockSpec((H, H), lambda p, i: (0, 0)),
            pl.BlockSpec((1, H), lambda p, i: (0, 0)),
            pl.BlockSpec((1, H), lambda p, i: (0, 0)),
            pl.BlockSpec((1, H), lambda p, i: (0, 0)),
            pl.BlockSpec((H, H), lambda p, i: (0, 0)),
            pl.BlockSpec((1, H), lambda p, i: (0, 0)),
            pl.BlockSpec((1, 1, BLK), lambda p, i: (i * p, 0, 0)),
        ],
        out_specs=[
            pl.BlockSpec((2, BLK, HALF), lambda p, i: (0, i * p, 0)),
            pl.BlockSpec((2, G, HALF), lambda p, i: (0, 0, 0)),
        ],
        out_shape=[
            jax.ShapeDtypeStruct((2, N_PAD, HALF), jnp.float32),
            jax.ShapeDtypeStruct((2, G, HALF), jnp.float32),
        ],
        scratch_shapes=[
            pltpu.VMEM((N, H), jnp.float32),
            pltpu.VMEM((1, H), jnp.float32),
            pltpu.VMEM((1, H), jnp.float32),
        ],
    )(hs, aggs, c['W1'], c['b1'].reshape(1, H), c['gamma'].reshape(1, H),
      c['beta'].reshape(1, H), c['W2'], c['b2'].reshape(1, H), batch3)


def _cls_body(p_ref, w1_ref, b1_ref, w2_ref, b2_ref, o_ref):
    acc = jnp.zeros((G, 3 * H), jnp.float32)
    for kk in range(6):
        acc = acc + jnp.dot(p_ref[kk], w1_ref[kk],
                            preferred_element_type=jnp.float32)
    hh = jnp.maximum(acc + b1_ref[...], 0.0)
    lg = jnp.dot(hh, w2_ref[...], preferred_element_type=jnp.float32) + b2_ref[...]
    m = jnp.max(lg, axis=1, keepdims=True)
    lse = m + jnp.log(jnp.sum(jnp.exp(lg - m), axis=1, keepdims=True))
    o_ref[...] = lg - lse


def _classifier(pcat, w1r, b1, w2p, b2p):
    return pl.pallas_call(
        _cls_body,
        grid=(1,),
        in_specs=[
            pl.BlockSpec((6, G, HALF), lambda i: (0, 0, 0)),
            pl.BlockSpec((6, HALF, 3 * H), lambda i: (0, 0, 0)),
            pl.BlockSpec((1, 3 * H), lambda i: (0, 0)),
            pl.BlockSpec((3 * H, 128), lambda i: (0, 0)),
            pl.BlockSpec((1, 128), lambda i: (0, 0)),
        ],
        out_specs=pl.BlockSpec((G, 128), lambda i: (0, 0)),
        out_shape=jax.ShapeDtypeStruct((G, 128), jnp.float32),
    )(pcat, w1r, b1, w2p, b2p)


def kernel(x, edge_index, batch, params):
    eidx32 = _pad_tables(edge_index, 2 * NTILES)
    eidx16 = _pad_tables(edge_index, NTILES)
    zrows = jnp.zeros((ROWS_PER_TILE, HALF), jnp.float32)
    batch3 = batch.reshape(NBLK, 1, BLK)
    c1, c2, c3 = params['c1'], params['c2'], params['c3']

    agg1 = _agg_l1(x, eidx32, zrows)
    h1, p1 = _gin_l1(x, agg1, c1, batch3)

    agg2 = _agg_l23(h1, eidx16, zrows)
    h2, p2 = _gin_l23(h1, agg2, c2, batch3)

    agg3 = _agg_l23(h2, eidx16, zrows)
    h3, p3 = _gin_l23(h2, agg3, c3, batch3)

    pcat = jnp.concatenate([p1, p2, p3], axis=0)
    w1r = params['lin1_W'].reshape(6, HALF, 3 * H)
    b1r = params['lin1_b'].reshape(1, 3 * H)
    w2p = jnp.pad(params['lin2_W'], ((0, 0), (0, 128 - C_OUT)))
    b2p = jnp.concatenate(
        [params['lin2_b'], jnp.full((128 - C_OUT,), -1e9, jnp.float32)]
    ).reshape(1, 128)
    out = _classifier(pcat, w1r, b1r, w2p, b2p)
    return out[:, :C_OUT]
